# Initial kernel scaffold; baseline (speedup 1.0000x reference)
import functools

import jax
import jax.numpy as jnp
from jax import lax
from jax.experimental import pallas as pl
from jax.experimental.pallas import tpu as pltpu
from jax.experimental.pallas import tpu_sc as plsc

_N = 10000
_E = 640000
_H = 128
_NODE1 = 83
_N1P = 96
_NLAYERS = 4

_NC = 2
_NS = 16
_NW = _NC * _NS
_CH = 128
_NCHUNK = 5024
_EP = _NCHUNK * _CH
_PER_W = _NCHUNK // _NW
_ROWS_S = _N // _NS

_BE = 2048
_BN = 1000
_BH = 500


def _sc_gather_add(wt):
    mesh = plsc.VectorSubcoreMesh(core_axis_name="c", subcore_axis_name="s")

    @functools.partial(
        pl.kernel,
        out_type=jax.ShapeDtypeStruct((_EP, wt), jnp.float32),
        mesh=mesh,
        scratch_types=[
            pltpu.VMEM((_CH,), jnp.int32),
            pltpu.VMEM((_CH,), jnp.int32),
            pltpu.VMEM((_CH, wt), jnp.float32),
            pltpu.VMEM((_CH, wt), jnp.float32),
            pltpu.SemaphoreType.DMA,
            pltpu.SemaphoreType.DMA,
        ],
    )
    def k(tab_a, tab_b, row_h, col_h, out_h, idx_a, idx_b, buf_a, buf_b,
          sem_a, sem_b):
        wid = lax.axis_index("s") * _NC + lax.axis_index("c")
        base = wid * _PER_W

        def chunk_body(j, carry):
            off = (base + j) * _CH
            pltpu.sync_copy(row_h.at[pl.ds(off, _CH)], idx_a)
            pltpu.sync_copy(col_h.at[pl.ds(off, _CH)], idx_b)
            ca = pltpu.async_copy(tab_a.at[idx_a], buf_a, sem_a)
            cb = pltpu.async_copy(tab_b.at[idx_b], buf_b, sem_b)
            ca.wait()
            cb.wait()

            def add_body(e, c2):
                for vv in range(wt // 16):
                    sl = pl.ds(vv * 16, 16)
                    buf_a[e, sl] = buf_a[e, sl] + buf_b[e, sl]
                return c2

            lax.fori_loop(0, _CH, add_body, 0)
            pltpu.sync_copy(buf_a, out_h.at[pl.ds(off, _CH)])
            return carry

        lax.fori_loop(0, _PER_W, chunk_body, 0)

    return k


def _make_sc_scatter():
    mesh = plsc.VectorSubcoreMesh(core_axis_name="c", subcore_axis_name="s")
    half = _NCHUNK // _NC
    per_s = half // _NS

    @functools.partial(
        pl.kernel,
        out_type=jax.ShapeDtypeStruct((_NC * _N, _H), jnp.float32),
        mesh=mesh,
        scratch_types=[
            pltpu.VMEM((_CH,), jnp.int32),
            pltpu.VMEM((_CH, _H), jnp.float32),
            pltpu.VMEM_SHARED((_N, _H), jnp.float32),
        ],
    )
    def k(m_h, row_h, zeros_h, out_h, idx, mbuf, agg):
        c = lax.axis_index("c")
        s = lax.axis_index("s")
        pltpu.sync_copy(zeros_h.at[pl.ds(s * _ROWS_S, _ROWS_S)],
                        agg.at[pl.ds(s * _ROWS_S, _ROWS_S)])
        plsc.subcore_barrier()
        base = c * half + s * per_s

        def chunk_body(j, carry):
            off = (base + j) * _CH
            pltpu.sync_copy(row_h.at[pl.ds(off, _CH)], idx)
            pltpu.sync_copy(m_h.at[pl.ds(off, _CH)], mbuf)
            pltpu.sync_copy(mbuf, agg.at[idx], add=True)
            return carry

        lax.fori_loop(0, per_s, chunk_body, 0)
        plsc.subcore_barrier()
        pltpu.sync_copy(agg.at[pl.ds(s * _ROWS_S, _ROWS_S)],
                        out_h.at[pl.ds(c * _N + s * _ROWS_S, _ROWS_S)])

    return k


def _run_sc_gather_add(tab_a, tab_b, row_p, col_p, wt):
    return _sc_gather_add(wt)(tab_a, tab_b, row_p, col_p)


def _run_sc_scatter(m, row_p, zeros_nh):
    return _make_sc_scatter()(m, row_p, zeros_nh)


def _full(shape):
    return pl.BlockSpec(shape, lambda i: tuple(0 for _ in shape))


def _dot(a, b):
    return jnp.dot(a, b, preferred_element_type=jnp.float32)


def _silu(x):
    return x * jax.nn.sigmoid(x)


def _tc_prep(h0p, w_emb_p, b_emb, we1a, be1v, we1b):
    def body(h0_ref, wemb_ref, bemb_ref, wa_ref, bav_ref, wb_ref,
             h_ref, a_ref, b_ref):
        h = _dot(h0_ref[...], wemb_ref[...]) + bemb_ref[...]
        h_ref[...] = h
        a_ref[...] = _dot(h, wa_ref[...]) + bav_ref[...]
        b_ref[...] = _dot(h, wb_ref[...])

    nblk = pl.BlockSpec((_BN, _H), lambda i: (i, 0))
    return pl.pallas_call(
        body,
        grid=(_N // _BN,),
        in_specs=[
            pl.BlockSpec((_BN, _N1P), lambda i: (i, 0)),
            _full((_N1P, _H)), _full((1, _H)),
            _full((_H, _H)), _full((1, _H)), _full((_H, _H)),
        ],
        out_specs=[nblk, nblk, nblk],
        out_shape=[jax.ShapeDtypeStruct((_N, _H), jnp.float32)] * 3,
    )(h0p, w_emb_p, b_emb, we1a, be1v, we1b)


def _edge_core(x, we2, be2v, wat, bav):
    s = _silu(x)
    m2 = _dot(s, we2) + be2v
    m = _silu(m2)
    att = jax.nn.sigmoid(jnp.sum(m * wat, axis=1, keepdims=True) + bav)
    return m * att


def _tc_edge0(m1g, ea_p, we2, be2v, wat, bav, wr, wev):
    def body(g_ref, ea_ref, we2_ref, be2_ref, wat_ref, bav_ref, wr_ref,
             wev_ref, m_ref, rad_ref):
        g = g_ref[...]
        cd = g[:, 128:131]
        rad = jnp.sum(cd * cd, axis=1, keepdims=True)
        x = g[:, :128] + rad * wr_ref[...] + ea_ref[...] * wev_ref[...]
        out = _edge_core(x, we2_ref[...], be2_ref[...], wat_ref[...],
                         bav_ref[...])
        eid = pl.program_id(0) * _BE + lax.broadcasted_iota(
            jnp.int32, (_BE, 1), 0)
        valid = eid < _E
        m_ref[...] = jnp.where(valid, out, 0.0)
        rad_ref[...] = jnp.where(valid, rad, 0.0)

    eblk = pl.BlockSpec((_BE, _H), lambda i: (i, 0))
    sblk = pl.BlockSpec((_BE, 1), lambda i: (i, 0))
    return pl.pallas_call(
        body,
        grid=(_EP // _BE,),
        in_specs=[
            pl.BlockSpec((_BE, 144), lambda i: (i, 0)), sblk,
            _full((_H, _H)), _full((1, _H)), _full((1, _H)), _full((1, 1)),
            _full((1, _H)), _full((1, _H)),
        ],
        out_specs=[eblk, sblk],
        out_shape=[jax.ShapeDtypeStruct((_EP, _H), jnp.float32),
                   jax.ShapeDtypeStruct((_EP, 1), jnp.float32)],
    )(m1g, ea_p, we2, be2v, wat, bav, wr, wev)


def _tc_edge(m1g, rad_p, ea_p, we2, be2v, wat, bav, wr, wev):
    def body(g_ref, rad_ref, ea_ref, we2_ref, be2_ref, wat_ref, bav_ref,
             wr_ref, wev_ref, m_ref):
        x = (g_ref[...] + rad_ref[...] * wr_ref[...]
             + ea_ref[...] * wev_ref[...])
        out = _edge_core(x, we2_ref[...], be2_ref[...], wat_ref[...],
                         bav_ref[...])
        eid = pl.program_id(0) * _BE + lax.broadcasted_iota(
            jnp.int32, (_BE, 1), 0)
        m_ref[...] = jnp.where(eid < _E, out, 0.0)

    eblk = pl.BlockSpec((_BE, _H), lambda i: (i, 0))
    sblk = pl.BlockSpec((_BE, 1), lambda i: (i, 0))
    return pl.pallas_call(
        body,
        grid=(_EP // _BE,),
        in_specs=[
            eblk, sblk, sblk,
            _full((_H, _H)), _full((1, _H)), _full((1, _H)), _full((1, 1)),
            _full((1, _H)), _full((1, _H)),
        ],
        out_specs=eblk,
        out_shape=jax.ShapeDtypeStruct((_EP, _H), jnp.float32),
    )(m1g, rad_p, ea_p, we2, be2v, wat, bav, wr, wev)


def _tc_node(h, agg0, agg1, h0p, wn1h, wn1a, wn1z, bn1v, wn2, bn2v,
             nxt=None):
    has_next = nxt is not None

    def body(h_ref, a0_ref, a1_ref, h0_ref, wn1h_ref, wn1a_ref, wn1z_ref,
             bn1_ref, wn2_ref, bn2_ref, *rest):
        if has_next:
            wea_ref, bea_ref, web_ref, h_out, a_out, b_out = rest
        else:
            (h_out,) = rest
        agg = a0_ref[...] + a1_ref[...]
        t = (_dot(h_ref[...], wn1h_ref[...]) + _dot(agg, wn1a_ref[...])
             + _dot(h0_ref[...], wn1z_ref[...]) + bn1_ref[...])
        hn = h_ref[...] + _dot(_silu(t), wn2_ref[...]) + bn2_ref[...]
        h_out[...] = hn
        if has_next:
            a_out[...] = _dot(hn, wea_ref[...]) + bea_ref[...]
            b_out[...] = _dot(hn, web_ref[...])

    nblk = pl.BlockSpec((_BN, _H), lambda i: (i, 0))
    in_specs = [
        nblk, nblk, nblk,
        pl.BlockSpec((_BN, _N1P), lambda i: (i, 0)),
        _full((_H, _H)), _full((_H, _H)), _full((_N1P, _H)), _full((1, _H)),
        _full((_H, _H)), _full((1, _H)),
    ]
    args = [h, agg0, agg1, h0p, wn1h, wn1a, wn1z, bn1v, wn2, bn2v]
    if has_next:
        wea, bea, web = nxt
        in_specs += [_full((_H, _H)), _full((1, _H)), _full((_H, _H))]
        args += [wea, bea, web]
        out_specs = [nblk, nblk, nblk]
        out_shape = [jax.ShapeDtypeStruct((_N, _H), jnp.float32)] * 3
    else:
        out_specs = [nblk]
        out_shape = [jax.ShapeDtypeStruct((_N, _H), jnp.float32)]
    return pl.pallas_call(
        body, grid=(_N // _BN,), in_specs=in_specs, out_specs=out_specs,
        out_shape=out_shape,
    )(*args)


def _tc_head(h, esm, wd1, bd1v, wd2, bd2v, wf1, bf1v, wf2, bf2v,
             wl1h, wl1e, bl1v, wl2t, bl2v):
    def body(h_ref, e_ref, wd1_ref, bd1_ref, wd2_ref, bd2_ref, wf1_ref,
             bf1_ref, wf2_ref, bf2_ref, wl1h_ref, wl1e_ref, bl1_ref,
             wl2_ref, bl2_ref, o_ref):
        hd = _dot(_silu(_dot(h_ref[...], wd1_ref[...]) + bd1_ref[...]),
                  wd2_ref[...]) + bd2_ref[...]
        e1 = jax.nn.relu(_dot(e_ref[...], wf1_ref[...]) + bf1_ref[...])
        e2 = jax.nn.relu(_dot(e1, wf2_ref[...]) + bf2_ref[...])
        f = _silu(_dot(hd, wl1h_ref[...]) + _dot(e2, wl1e_ref[...])
                  + bl1_ref[...])
        o = jnp.sum(f * wl2_ref[...], axis=1, keepdims=True) + bl2_ref[...]
        o_ref[...] = jax.nn.sigmoid(o)

    return pl.pallas_call(
        body,
        grid=(_N // _BH,),
        in_specs=[
            pl.BlockSpec((_BH, _H), lambda i: (i, 0)),
            pl.BlockSpec((_BH, 1280), lambda i: (i, 0)),
            _full((_H, _H)), _full((1, _H)), _full((_H, _H)), _full((1, _H)),
            _full((1280, 256)), _full((1, 256)), _full((256, _H)),
            _full((1, _H)), _full((_H, 256)), _full((_H, 256)),
            _full((1, 256)), _full((1, 256)), _full((1, 1)),
        ],
        out_specs=pl.BlockSpec((_BH, 1), lambda i: (i, 0)),
        out_shape=jax.ShapeDtypeStruct((_N, 1), jnp.float32),
    )(h, esm, wd1, bd1v, wd2, bd2v, wf1, bf1v, wf2, bf2v, wl1h, wl1e,
      bl1v, wl2t, bl2v)


def kernel(node_attrs, coords, edge_index, edge_attrs, W_emb, b_emb, We1,
           be1, We2, be2, Wa, ba, Wn1, bn1, Wn2, bn2, Wd1, bd1, Wd2, bd2,
           Wf1, bf1, Wf2, bf2, Wl1, bl1, Wl2, bl2):
    f32 = jnp.float32
    h0p = jnp.pad(node_attrs[:, :_NODE1], ((0, 0), (0, _N1P - _NODE1)))
    esm_in = node_attrs[:, _NODE1:]
    w_emb_p = jnp.pad(W_emb, ((0, _N1P - _NODE1), (0, 0)))

    pad_e = _EP - _E
    row_p = jnp.pad(edge_index[0], (0, pad_e))
    col_p = jnp.pad(edge_index[1], (0, pad_e))
    ea_p = jnp.pad(edge_attrs, (0, pad_e)).reshape(_EP, 1)
    zeros_nh = jnp.zeros((_N, _H), f32)

    def v(x):
        return x.reshape(1, -1)

    h, a_tab, b_tab = _tc_prep(h0p, w_emb_p, v(b_emb), We1[0, :_H],
                               v(be1[0]), We1[0, _H:2 * _H])

    coords_a = jnp.pad(coords, ((0, 0), (0, 13)))
    coords_b = jnp.pad(-coords, ((0, 0), (0, 13)))

    rad_p = None
    for i in range(_NLAYERS):
        wr, wev = v(We1[i, 2 * _H]), v(We1[i, 2 * _H + 1])
        wat, bav = v(Wa[i][:, 0]), ba[i].reshape(1, 1)
        if i == 0:
            tab_a0 = jnp.concatenate([a_tab, coords_a], axis=1)
            tab_b0 = jnp.concatenate([b_tab, coords_b], axis=1)
            m1g = _run_sc_gather_add(tab_a0, tab_b0, row_p, col_p, 144)
            m, rad_p = _tc_edge0(m1g, ea_p, We2[i], v(be2[i]), wat, bav,
                                 wr, wev)
        else:
            m1g = _run_sc_gather_add(a_tab, b_tab, row_p, col_p, _H)
            m = _tc_edge(m1g, rad_p, ea_p, We2[i], v(be2[i]), wat, bav,
                         wr, wev)
        aggp = _run_sc_scatter(m, row_p, zeros_nh)
        agg0, agg1 = aggp[:_N], aggp[_N:]
        wn1h, wn1a = Wn1[i, :_H], Wn1[i, _H:2 * _H]
        wn1z = jnp.pad(Wn1[i, 2 * _H:], ((0, _N1P - _NODE1), (0, 0)))
        if i + 1 < _NLAYERS:
            nxt = (We1[i + 1, :_H], v(be1[i + 1]), We1[i + 1, _H:2 * _H])
            h, a_tab, b_tab = _tc_node(h, agg0, agg1, h0p, wn1h, wn1a,
                                       wn1z, v(bn1[i]), Wn2[i], v(bn2[i]),
                                       nxt=nxt)
        else:
            (h,) = _tc_node(h, agg0, agg1, h0p, wn1h, wn1a, wn1z,
                            v(bn1[i]), Wn2[i], v(bn2[i]))

    return _tc_head(h, esm_in, Wd1, v(bd1), Wd2, v(bd2), Wf1, v(bf1),
                    Wf2, v(bf2), Wl1[:_H], Wl1[_H:], v(bl1),
                    v(Wl2[:, 0]), bl2.reshape(1, 1))


# speedup vs baseline: 3.9324x; 3.9324x over previous
import functools

import jax
import jax.numpy as jnp
from jax import lax
from jax.experimental import pallas as pl
from jax.experimental.pallas import tpu as pltpu
from jax.experimental.pallas import tpu_sc as plsc

_N = 10000
_E = 640000
_H = 128
_NODE1 = 83
_N1P = 96
_NLAYERS = 4

_NC = 2
_NS = 16
_NW = _NC * _NS
_CH = 128
_NCHUNK = 5024
_EP = _NCHUNK * _CH
_PER_W = _NCHUNK // _NW
_NP = 10240
_ROWS_S = _NP // _NS

_BE = 2048
_BN = 1000
_BH = 1000


def _sc_gather_add(with_radial):
    mesh = plsc.VectorSubcoreMesh(core_axis_name="c", subcore_axis_name="s")

    out_type = [jax.ShapeDtypeStruct((_EP, _H), jnp.float32)]
    scratch = [
        pltpu.VMEM((_CH,), jnp.int32),
        pltpu.VMEM((_CH,), jnp.int32),
        pltpu.VMEM((_CH, _H), jnp.float32),
        pltpu.VMEM((_CH, _H), jnp.float32),
        pltpu.SemaphoreType.DMA,
        pltpu.SemaphoreType.DMA,
    ]
    if with_radial:
        out_type = out_type + [jax.ShapeDtypeStruct((_NCHUNK, _CH),
                                                    jnp.float32)]
        scratch = scratch + [
            pltpu.VMEM((3 * _N,), jnp.float32),
            pltpu.VMEM((_CH,), jnp.float32),
        ]

    @functools.partial(
        pl.kernel, out_type=out_type, mesh=mesh, scratch_types=scratch,
        compiler_params=pltpu.CompilerParams(needs_layout_passes=False))
    def k(*refs):
        if with_radial:
            (tab_a, tab_b, row_h, col_h, coords_h, out_h, rad_h,
             idx_a, idx_b, buf_a, buf_b, sem_a, sem_b, cbuf, radbuf) = refs
            pltpu.sync_copy(coords_h, cbuf)
        else:
            (tab_a, tab_b, row_h, col_h, out_h,
             idx_a, idx_b, buf_a, buf_b, sem_a, sem_b) = refs
        wid = lax.axis_index("s") * _NC + lax.axis_index("c")
        base = wid * _PER_W

        def chunk_body(j, carry):
            cc = base + j
            off = cc * _CH
            pltpu.sync_copy(row_h.at[pl.ds(off, _CH)], idx_a)
            pltpu.sync_copy(col_h.at[pl.ds(off, _CH)], idx_b)
            ca = pltpu.async_copy(tab_a.at[idx_a], buf_a, sem_a)
            cb = pltpu.async_copy(tab_b.at[idx_b], buf_b, sem_b)
            if with_radial:
                for g in range(_CH // 16):
                    sl = pl.ds(g * 16, 16)
                    ra = idx_a[sl] * 3
                    rb = idx_b[sl] * 3
                    acc = None
                    for dim in range(3):
                        d = (plsc.load_gather(cbuf, [ra + dim])
                             - plsc.load_gather(cbuf, [rb + dim]))
                        d2 = d * d
                        acc = d2 if acc is None else acc + d2
                    radbuf[sl] = acc
                pltpu.sync_copy(radbuf, rad_h.at[cc])
            ca.wait()
            cb.wait()

            def add_body(e, c2):
                for vv in range(_H // 16):
                    sl = pl.ds(vv * 16, 16)
                    buf_a[e, sl] = buf_a[e, sl] + buf_b[e, sl]
                return c2

            lax.fori_loop(0, _CH, add_body, 0)
            pltpu.sync_copy(buf_a, out_h.at[pl.ds(off, _CH)])
            return carry

        lax.fori_loop(0, _PER_W, chunk_body, 0)

    return k


def _make_sc_scatter():
    mesh = plsc.VectorSubcoreMesh(core_axis_name="c", subcore_axis_name="s")
    half = _NCHUNK // _NC
    per_s = half // _NS

    @functools.partial(
        pl.kernel,
        out_type=jax.ShapeDtypeStruct((_NC * _NP, _H), jnp.float32),
        mesh=mesh,
        scratch_types=[
            pltpu.VMEM((_CH,), jnp.int32),
            pltpu.VMEM((_CH, _H), jnp.float32),
            pltpu.VMEM_SHARED((_NP, _H), jnp.float32),
        ],
    )
    def k(m_h, row_h, zeros_h, out_h, idx, mbuf, agg):
        c = lax.axis_index("c")
        s = lax.axis_index("s")
        pltpu.sync_copy(zeros_h.at[pl.ds(s * _ROWS_S, _ROWS_S)],
                        agg.at[pl.ds(s * _ROWS_S, _ROWS_S)])
        plsc.subcore_barrier()
        base = c * half + s * per_s

        def chunk_body(j, carry):
            off = (base + j) * _CH
            pltpu.sync_copy(row_h.at[pl.ds(off, _CH)], idx)
            pltpu.sync_copy(m_h.at[pl.ds(off, _CH)], mbuf)
            pltpu.sync_copy(mbuf, agg.at[idx], add=True)
            return carry

        lax.fori_loop(0, per_s, chunk_body, 0)
        plsc.subcore_barrier()
        pltpu.sync_copy(agg.at[pl.ds(s * _ROWS_S, _ROWS_S)],
                        out_h.at[pl.ds(c * _NP + s * _ROWS_S, _ROWS_S)])

    return k


def _run_sc_gather_add(tab_a, tab_b, row_p, col_p, coords_flat=None):
    if coords_flat is None:
        return _sc_gather_add(False)(tab_a, tab_b, row_p, col_p)[0]
    return _sc_gather_add(True)(tab_a, tab_b, row_p, col_p, coords_flat)


def _run_sc_scatter(m, row_p, zeros_nh):
    return _make_sc_scatter()(m, row_p, zeros_nh)


def _full(shape):
    return pl.BlockSpec(shape, lambda i: tuple(0 for _ in shape))


def _dot(a, b):
    return jnp.dot(a, b, preferred_element_type=jnp.float32)


def _silu(x):
    return x * jax.nn.sigmoid(x)


def _tc_prep(h0p, w_emb_p, b_emb, we1a, be1v, we1b):
    def body(h0_ref, wemb_ref, bemb_ref, wa_ref, bav_ref, wb_ref,
             h_ref, a_ref, b_ref):
        h = _dot(h0_ref[...], wemb_ref[...]) + bemb_ref[...]
        h_ref[...] = h
        a_ref[...] = _dot(h, wa_ref[...]) + bav_ref[...]
        b_ref[...] = _dot(h, wb_ref[...])

    nblk = pl.BlockSpec((_BN, _H), lambda i: (i, 0))
    return pl.pallas_call(
        body,
        grid=(_N // _BN,),
        in_specs=[
            pl.BlockSpec((_BN, _N1P), lambda i: (i, 0)),
            _full((_N1P, _H)), _full((1, _H)),
            _full((_H, _H)), _full((1, _H)), _full((_H, _H)),
        ],
        out_specs=[nblk, nblk, nblk],
        out_shape=[jax.ShapeDtypeStruct((_N, _H), jnp.float32)] * 3,
    )(h0p, w_emb_p, b_emb, we1a, be1v, we1b)


def _edge_core(x, we2, be2v, wat, bav):
    s = _silu(x)
    m2 = _dot(s, we2) + be2v
    m = _silu(m2)
    att = jax.nn.sigmoid(jnp.sum(m * wat, axis=1, keepdims=True) + bav)
    return m * att


def _tc_edge(m1g, rad_p, ea_p, we2, be2v, wat, bav, wr, wev):
    def body(g_ref, rad_ref, ea_ref, we2_ref, be2_ref, wat_ref, bav_ref,
             wr_ref, wev_ref, m_ref):
        x = (g_ref[...] + rad_ref[...] * wr_ref[...]
             + ea_ref[...] * wev_ref[...])
        out = _edge_core(x, we2_ref[...], be2_ref[...], wat_ref[...],
                         bav_ref[...])
        eid = pl.program_id(0) * _BE + lax.broadcasted_iota(
            jnp.int32, (_BE, 1), 0)
        m_ref[...] = jnp.where(eid < _E, out, 0.0)

    eblk = pl.BlockSpec((_BE, _H), lambda i: (i, 0))
    sblk = pl.BlockSpec((_BE, 1), lambda i: (i, 0))
    return pl.pallas_call(
        body,
        grid=(_EP // _BE,),
        in_specs=[
            eblk, sblk, sblk,
            _full((_H, _H)), _full((1, _H)), _full((1, _H)), _full((1, 1)),
            _full((1, _H)), _full((1, _H)),
        ],
        out_specs=eblk,
        out_shape=jax.ShapeDtypeStruct((_EP, _H), jnp.float32),
    )(m1g, rad_p, ea_p, we2, be2v, wat, bav, wr, wev)


def _tc_node(h, agg0, agg1, h0p, wn1h, wn1a, wn1z, bn1v, wn2, bn2v,
             nxt=None):
    has_next = nxt is not None

    def body(h_ref, a0_ref, a1_ref, h0_ref, wn1h_ref, wn1a_ref, wn1z_ref,
             bn1_ref, wn2_ref, bn2_ref, *rest):
        if has_next:
            wea_ref, bea_ref, web_ref, h_out, a_out, b_out = rest
        else:
            (h_out,) = rest
        agg = a0_ref[...] + a1_ref[...]
        t = (_dot(h_ref[...], wn1h_ref[...]) + _dot(agg, wn1a_ref[...])
             + _dot(h0_ref[...], wn1z_ref[...]) + bn1_ref[...])
        hn = h_ref[...] + _dot(_silu(t), wn2_ref[...]) + bn2_ref[...]
        h_out[...] = hn
        if has_next:
            a_out[...] = _dot(hn, wea_ref[...]) + bea_ref[...]
            b_out[...] = _dot(hn, web_ref[...])

    nblk = pl.BlockSpec((_BN, _H), lambda i: (i, 0))
    in_specs = [
        nblk, nblk, nblk,
        pl.BlockSpec((_BN, _N1P), lambda i: (i, 0)),
        _full((_H, _H)), _full((_H, _H)), _full((_N1P, _H)), _full((1, _H)),
        _full((_H, _H)), _full((1, _H)),
    ]
    args = [h, agg0, agg1, h0p, wn1h, wn1a, wn1z, bn1v, wn2, bn2v]
    if has_next:
        wea, bea, web = nxt
        in_specs += [_full((_H, _H)), _full((1, _H)), _full((_H, _H))]
        args += [wea, bea, web]
        out_specs = [nblk, nblk, nblk]
        out_shape = [jax.ShapeDtypeStruct((_N, _H), jnp.float32)] * 3
    else:
        out_specs = [nblk]
        out_shape = [jax.ShapeDtypeStruct((_N, _H), jnp.float32)]
    return pl.pallas_call(
        body, grid=(_N // _BN,), in_specs=in_specs, out_specs=out_specs,
        out_shape=out_shape,
    )(*args)


def _tc_head(h, esm, wd1, bd1v, wd2, bd2v, wf1, bf1v, wf2, bf2v,
             wl1h, wl1e, bl1v, wl2t, bl2v):
    def body(h_ref, e_ref, wd1_ref, bd1_ref, wd2_ref, bd2_ref, wf1_ref,
             bf1_ref, wf2_ref, bf2_ref, wl1h_ref, wl1e_ref, bl1_ref,
             wl2_ref, bl2_ref, o_ref):
        hd = _dot(_silu(_dot(h_ref[...], wd1_ref[...]) + bd1_ref[...]),
                  wd2_ref[...]) + bd2_ref[...]
        e1 = jax.nn.relu(_dot(e_ref[...], wf1_ref[...]) + bf1_ref[...])
        e2 = jax.nn.relu(_dot(e1, wf2_ref[...]) + bf2_ref[...])
        f = _silu(_dot(hd, wl1h_ref[...]) + _dot(e2, wl1e_ref[...])
                  + bl1_ref[...])
        o = jnp.sum(f * wl2_ref[...], axis=1, keepdims=True) + bl2_ref[...]
        o_ref[...] = jax.nn.sigmoid(o)

    return pl.pallas_call(
        body,
        grid=(_N // _BH,),
        in_specs=[
            pl.BlockSpec((_BH, _H), lambda i: (i, 0)),
            pl.BlockSpec((_BH, 1280), lambda i: (i, 0)),
            _full((_H, _H)), _full((1, _H)), _full((_H, _H)), _full((1, _H)),
            _full((1280, 256)), _full((1, 256)), _full((256, _H)),
            _full((1, _H)), _full((_H, 256)), _full((_H, 256)),
            _full((1, 256)), _full((1, 256)), _full((1, 1)),
        ],
        out_specs=pl.BlockSpec((_BH, 1), lambda i: (i, 0)),
        out_shape=jax.ShapeDtypeStruct((_N, 1), jnp.float32),
    )(h, esm, wd1, bd1v, wd2, bd2v, wf1, bf1v, wf2, bf2v, wl1h, wl1e,
      bl1v, wl2t, bl2v)


def kernel(node_attrs, coords, edge_index, edge_attrs, W_emb, b_emb, We1,
           be1, We2, be2, Wa, ba, Wn1, bn1, Wn2, bn2, Wd1, bd1, Wd2, bd2,
           Wf1, bf1, Wf2, bf2, Wl1, bl1, Wl2, bl2):
    f32 = jnp.float32
    h0p = jnp.pad(node_attrs[:, :_NODE1], ((0, 0), (0, _N1P - _NODE1)))
    esm_in = node_attrs[:, _NODE1:]
    w_emb_p = jnp.pad(W_emb, ((0, _N1P - _NODE1), (0, 0)))

    pad_e = _EP - _E
    row_p = jnp.pad(edge_index[0], (0, pad_e))
    col_p = jnp.pad(edge_index[1], (0, pad_e))
    ea_p = jnp.pad(edge_attrs, (0, pad_e)).reshape(_EP, 1)
    zeros_nh = jnp.zeros((_NP, _H), f32)

    def v(x):
        return x.reshape(1, -1)

    h, a_tab, b_tab = _tc_prep(h0p, w_emb_p, v(b_emb), We1[0, :_H],
                               v(be1[0]), We1[0, _H:2 * _H])

    coords_flat = coords.reshape(-1)

    rad_p = None
    for i in range(_NLAYERS):
        wr, wev = v(We1[i, 2 * _H]), v(We1[i, 2 * _H + 1])
        wat, bav = v(Wa[i][:, 0]), ba[i].reshape(1, 1)
        if i == 0:
            m1g, rad2d = _run_sc_gather_add(a_tab, b_tab, row_p, col_p,
                                            coords_flat)
            rad_p = rad2d.reshape(_EP, 1)
        else:
            m1g = _run_sc_gather_add(a_tab, b_tab, row_p, col_p)
        m = _tc_edge(m1g, rad_p, ea_p, We2[i], v(be2[i]), wat, bav,
                     wr, wev)
        aggp = _run_sc_scatter(m, row_p, zeros_nh)
        agg0, agg1 = aggp[:_N], aggp[_NP:_NP + _N]
        wn1h, wn1a = Wn1[i, :_H], Wn1[i, _H:2 * _H]
        wn1z = jnp.pad(Wn1[i, 2 * _H:], ((0, _N1P - _NODE1), (0, 0)))
        if i + 1 < _NLAYERS:
            nxt = (We1[i + 1, :_H], v(be1[i + 1]), We1[i + 1, _H:2 * _H])
            h, a_tab, b_tab = _tc_node(h, agg0, agg1, h0p, wn1h, wn1a,
                                       wn1z, v(bn1[i]), Wn2[i], v(bn2[i]),
                                       nxt=nxt)
        else:
            (h,) = _tc_node(h, agg0, agg1, h0p, wn1h, wn1a, wn1z,
                            v(bn1[i]), Wn2[i], v(bn2[i]))

    return _tc_head(h, esm_in, Wd1, v(bd1), Wd2, v(bd2), Wf1, v(bf1),
                    Wf2, v(bf2), Wl1[:_H], Wl1[_H:], v(bl1),
                    v(Wl2[:, 0]), bl2.reshape(1, 1))
